# Initial kernel scaffold; baseline (speedup 1.0000x reference)
import jax
import jax.numpy as jnp
from jax.experimental import pallas as pl


def kernel(x, edge_index, edge_attr, batch, params):
    raise NotImplementedError("write your pallas kernel here")


# speedup vs baseline: 2.2786x; 2.2786x over previous
import functools

import jax
import jax.numpy as jnp
from jax import lax
from jax.experimental import pallas as pl
from jax.experimental.pallas import tpu as pltpu
from jax.experimental.pallas import tpu_sc as plsc

H = 128
N = 10000
NP = 10240
E = 320000
NLAYERS = 4
EPS_GEN = 1e-7
LN_EPS = 1e-5

NWORKERS = 32
DPT = H // NWORKERS
CHUNK = 1280
NCHUNK = E // CHUNK
GROUPS = CHUNK // 16
PVLEN = 3 * H


def _sc_edge_pass(h_flat, src, dst, attr, pvec):
    mesh = plsc.VectorSubcoreMesh(core_axis_name="c", subcore_axis_name="s")

    @functools.partial(
        pl.kernel,
        mesh=mesh,
        compiler_params=pltpu.CompilerParams(needs_layout_passes=False),
        out_type=[
            jax.ShapeDtypeStruct((H * NP,), jnp.float32),
            jax.ShapeDtypeStruct((H * NP,), jnp.float32),
        ],
        scratch_types=[
            pltpu.VMEM((DPT * NP,), jnp.float32),
            pltpu.VMEM((DPT * NP,), jnp.float32),
            pltpu.VMEM((DPT * NP,), jnp.float32),
            pltpu.VMEM((CHUNK,), jnp.int32),
            pltpu.VMEM((CHUNK,), jnp.int32),
            pltpu.VMEM((CHUNK,), jnp.float32),
            pltpu.VMEM((PVLEN,), jnp.float32),
        ],
    )
    def k(h_hbm, src_hbm, dst_hbm, attr_hbm, pv_hbm, num_hbm, den_hbm,
          hcols, numv, denv, srcb, dstb, attrb, pv):
        wid = lax.axis_index("s") * 2 + lax.axis_index("c")
        base = wid * (DPT * NP)
        pltpu.sync_copy(h_hbm.at[pl.ds(base, DPT * NP)], hcols)
        pltpu.sync_copy(pv_hbm, pv)

        zz = jnp.zeros((16,), jnp.float32)

        def zbody(j, carry):
            numv[pl.ds(j * 16, 16)] = zz
            denv[pl.ds(j * 16, 16)] = zz
            return carry

        lax.fori_loop(0, DPT * NP // 16, zbody, 0)

        zi = jnp.zeros((16,), jnp.int32)
        tvec = pv[pl.ds(2 * H, 16)]
        ewd = [plsc.load_gather(pv, [zi + (wid * DPT + d)]) for d in range(DPT)]
        ebd = [plsc.load_gather(pv, [zi + (H + wid * DPT + d)]) for d in range(DPT)]

        def chunk_body(c, carry):
            off = c * CHUNK
            pltpu.sync_copy(src_hbm.at[pl.ds(off, CHUNK)], srcb)
            pltpu.sync_copy(dst_hbm.at[pl.ds(off, CHUNK)], dstb)
            pltpu.sync_copy(attr_hbm.at[pl.ds(off, CHUNK)], attrb)

            def gbody(g, carry2):
                s16 = srcb[pl.ds(g * 16, 16)]
                d16 = dstb[pl.ds(g * 16, 16)]
                a16 = attrb[pl.ds(g * 16, 16)]
                for d in range(DPT):
                    hv = plsc.load_gather(hcols, [s16 + d * NP])
                    msg = jnp.maximum(hv + (a16 * ewd[d] + ebd[d]), 0.0) + EPS_GEN
                    ex = jnp.exp(msg * tvec)
                    plsc.addupdate_scatter(numv, [d16 + d * NP], msg * ex)
                    plsc.addupdate_scatter(denv, [d16 + d * NP], ex)
                return carry2

            lax.fori_loop(0, GROUPS, gbody, 0)
            return carry

        lax.fori_loop(0, NCHUNK, chunk_body, 0)

        pltpu.sync_copy(numv, num_hbm.at[pl.ds(base, DPT * NP)])
        pltpu.sync_copy(denv, den_hbm.at[pl.ds(base, DPT * NP)])

    return k(h_flat, src, dst, attr, pvec)


NT = 1024
GRID = NP // NT


def _ln_cols(y, g, b):
    mu = jnp.mean(y, axis=0, keepdims=True)
    var = jnp.mean((y - mu) ** 2, axis=0, keepdims=True)
    return (y - mu) / jnp.sqrt(var + LN_EPS) * g + b


def _encode_body(x_ref, nw_ref, nb_ref, o_ref):
    o_ref[...] = nw_ref[...] * x_ref[...] + nb_ref[...]


def _tc_encode(x2, nw, nb):
    return pl.pallas_call(
        _encode_body,
        grid=(GRID,),
        in_specs=[
            pl.BlockSpec((1, NT), lambda i: (0, i)),
            pl.BlockSpec((H, 1), lambda i: (0, 0)),
            pl.BlockSpec((H, 1), lambda i: (0, 0)),
        ],
        out_specs=pl.BlockSpec((H, NT), lambda i: (0, i)),
        out_shape=jax.ShapeDtypeStruct((H, NP), jnp.float32),
    )(x2, nw, nb)


def _layer_body(residual, num_ref, den_ref, hn_ref, hp_ref, w1_ref, b1_ref,
                g1_ref, be1_ref, w2_ref, b2_ref, gn_ref, bn_ref,
                ho_ref, hn2_ref):
    agg = num_ref[...] / (den_ref[...] + 1e-16)
    outt = agg + hn_ref[...]
    y1 = jnp.dot(w1_ref[...], outt, preferred_element_type=jnp.float32)
    y1 = y1 + b1_ref[...]
    y1 = jnp.maximum(_ln_cols(y1, g1_ref[...], be1_ref[...]), 0.0)
    y2 = jnp.dot(w2_ref[...], y1, preferred_element_type=jnp.float32)
    y2 = y2 + b2_ref[...]
    h_new = hp_ref[...] + y2 if residual else y2
    ho_ref[...] = h_new
    hn2_ref[...] = jnp.maximum(_ln_cols(h_new, gn_ref[...], bn_ref[...]), 0.0)


def _tc_layer(num2, den2, hn2, hp2, w1t, b1c, g1c, be1c, w2t, b2c, gnc, bnc,
              residual):
    big = pl.BlockSpec((H, NT), lambda i: (0, i))
    return pl.pallas_call(
        functools.partial(_layer_body, residual),
        grid=(GRID,),
        in_specs=[
            big, big, big, big,
            pl.BlockSpec((2 * H, H), lambda i: (0, 0)),
            pl.BlockSpec((2 * H, 1), lambda i: (0, 0)),
            pl.BlockSpec((2 * H, 1), lambda i: (0, 0)),
            pl.BlockSpec((2 * H, 1), lambda i: (0, 0)),
            pl.BlockSpec((H, 2 * H), lambda i: (0, 0)),
            pl.BlockSpec((H, 1), lambda i: (0, 0)),
            pl.BlockSpec((H, 1), lambda i: (0, 0)),
            pl.BlockSpec((H, 1), lambda i: (0, 0)),
        ],
        out_specs=[big, big],
        out_shape=[
            jax.ShapeDtypeStruct((H, NP), jnp.float32),
            jax.ShapeDtypeStruct((H, NP), jnp.float32),
        ],
    )(num2, den2, hn2, hp2, w1t, b1c, g1c, be1c, w2t, b2c, gnc, bnc)


def _last_body(num_ref, den_ref, hn_ref, hp_ref, w1_ref, b1_ref, g1_ref,
               be1_ref, w2_ref, b2_ref, gn_ref, bn_ref, lw_ref, sc_ref):
    agg = num_ref[...] / (den_ref[...] + 1e-16)
    outt = agg + hn_ref[...]
    y1 = jnp.dot(w1_ref[...], outt, preferred_element_type=jnp.float32)
    y1 = y1 + b1_ref[...]
    y1 = jnp.maximum(_ln_cols(y1, g1_ref[...], be1_ref[...]), 0.0)
    y2 = jnp.dot(w2_ref[...], y1, preferred_element_type=jnp.float32)
    y2 = y2 + b2_ref[...]
    h_new = hp_ref[...] + y2
    hfin = jnp.maximum(_ln_cols(h_new, gn_ref[...], bn_ref[...]), 0.0)
    part = jnp.sum(hfin * lw_ref[...]).reshape(1, 1)

    @pl.when(pl.program_id(0) == 0)
    def _():
        sc_ref[...] = jnp.zeros((1, 1), jnp.float32)

    sc_ref[...] += part


def _tc_last(num2, den2, hn2, hp2, w1t, b1c, g1c, be1c, w2t, b2c, gnc, bnc,
             lwt):
    big = pl.BlockSpec((H, NT), lambda i: (0, i))
    return pl.pallas_call(
        _last_body,
        grid=(GRID,),
        in_specs=[
            big, big, big, big,
            pl.BlockSpec((2 * H, H), lambda i: (0, 0)),
            pl.BlockSpec((2 * H, 1), lambda i: (0, 0)),
            pl.BlockSpec((2 * H, 1), lambda i: (0, 0)),
            pl.BlockSpec((2 * H, 1), lambda i: (0, 0)),
            pl.BlockSpec((H, 2 * H), lambda i: (0, 0)),
            pl.BlockSpec((H, 1), lambda i: (0, 0)),
            pl.BlockSpec((H, 1), lambda i: (0, 0)),
            pl.BlockSpec((H, 1), lambda i: (0, 0)),
            big,
        ],
        out_specs=pl.BlockSpec((1, 1), lambda i: (0, 0)),
        out_shape=jax.ShapeDtypeStruct((1, 1), jnp.float32),
    )(num2, den2, hn2, hp2, w1t, b1c, g1c, be1c, w2t, b2c, gnc, bnc, lwt)


def kernel(x, edge_index, edge_attr, batch, params):
    src = edge_index[0]
    dst = edge_index[1]
    ew = params['edge_w'][0]
    eb = params['edge_b']
    nw = params['node_w'][0].reshape(H, 1)
    nb = params['node_b'].reshape(H, 1)
    lwt = jnp.pad(params['lin_w'].reshape(N, H).T, ((0, 0), (0, NP - N)))

    pvecs = [
        jnp.concatenate([ew, eb, jnp.broadcast_to(params['t'][i], (16,)),
                         jnp.zeros((PVLEN - 2 * H - 16,), jnp.float32)])
        for i in range(NLAYERS)
    ]
    w1t = [params['w1'][i].T for i in range(NLAYERS)]
    w2t = [params['w2'][i].T for i in range(NLAYERS)]
    b1c = [params['b1'][i].reshape(2 * H, 1) for i in range(NLAYERS)]
    g1c = [params['g1'][i].reshape(2 * H, 1) for i in range(NLAYERS)]
    be1c = [params['be1'][i].reshape(2 * H, 1) for i in range(NLAYERS)]
    b2c = [params['b2'][i].reshape(H, 1) for i in range(NLAYERS)]
    bg = [params['blk_g'][i].reshape(H, 1) for i in range(NLAYERS)]
    bb = [params['blk_b'][i].reshape(H, 1) for i in range(NLAYERS)]

    hT = _tc_encode(jnp.pad(x.reshape(1, N), ((0, 0), (0, NP - N))), nw, nb)

    h = hT
    hn = hT
    for i in range(NLAYERS):
        num_f, den_f = _sc_edge_pass(hn.reshape(H * NP), src, dst, edge_attr,
                                     pvecs[i])
        num2 = num_f.reshape(H, NP)
        den2 = den_f.reshape(H, NP)
        if i < NLAYERS - 1:
            h, hn = _tc_layer(num2, den2, hn, h, w1t[i], b1c[i], g1c[i],
                              be1c[i], w2t[i], b2c[i], bg[i + 1], bb[i + 1],
                              residual=(i > 0))
        else:
            scal = _tc_last(num2, den2, hn, h, w1t[i], b1c[i], g1c[i],
                            be1c[i], w2t[i], b2c[i], bg[0], bb[0], lwt)

    return scal.reshape(1) + params['lin_b']


# speedup vs baseline: 8.2111x; 3.6036x over previous
import functools

import jax
import jax.numpy as jnp
from jax import lax
from jax.experimental import pallas as pl
from jax.experimental.pallas import tpu as pltpu
from jax.experimental.pallas import tpu_sc as plsc

H = 128
N = 10000
NP = 10240
E = 320000
NLAYERS = 4
EPS_GEN = 1e-7
LN_EPS = 1e-5

NWORKERS = 32
DPT = H // NWORKERS
CHUNK = 800
NCHUNK = E // CHUNK
GROUPS = CHUNK // 16
PVLEN = 3 * H


def _sc_edge_pass(h_flat, epacked, pvec):
    mesh = plsc.VectorSubcoreMesh(core_axis_name="c", subcore_axis_name="s")

    @functools.partial(
        pl.kernel,
        mesh=mesh,
        compiler_params=pltpu.CompilerParams(needs_layout_passes=False),
        out_type=[
            jax.ShapeDtypeStruct((H * NP,), jnp.float32),
            jax.ShapeDtypeStruct((H * NP,), jnp.float32),
        ],
        scratch_types=[
            pltpu.VMEM((DPT * NP,), jnp.float32),
            pltpu.VMEM((DPT * NP,), jnp.float32),
            pltpu.VMEM((DPT * NP,), jnp.float32),
            pltpu.VMEM((3 * CHUNK,), jnp.int32),
            pltpu.VMEM((3 * CHUNK,), jnp.int32),
            pltpu.VMEM((PVLEN,), jnp.float32),
            pltpu.SemaphoreType.DMA,
            pltpu.SemaphoreType.DMA,
        ],
    )
    def k(h_hbm, e_hbm, pv_hbm, num_hbm, den_hbm,
          hcols, numv, denv, eb0, eb1, pv, sem0, sem1):
        wid = lax.axis_index("s") * 2 + lax.axis_index("c")
        base = wid * (DPT * NP)
        pltpu.sync_copy(h_hbm.at[pl.ds(base, DPT * NP)], hcols)
        pltpu.sync_copy(pv_hbm, pv)

        zz = jnp.zeros((16,), jnp.float32)

        @plsc.parallel_loop(0, DPT * NP // 16, unroll=8)
        def _(j):
            numv[pl.ds(j * 16, 16)] = zz
            denv[pl.ds(j * 16, 16)] = zz

        zi = jnp.zeros((16,), jnp.int32)
        tvec = pv[pl.ds(2 * H, 16)]
        ewd = [plsc.load_gather(pv, [zi + (wid * DPT + d)]) for d in range(DPT)]
        ebd = [plsc.load_gather(pv, [zi + (H + wid * DPT + d)]) for d in range(DPT)]

        ebufs = (eb0, eb1)
        sems = (sem0, sem1)

        def process(ebuf):
            @plsc.parallel_loop(0, GROUPS, unroll=4)
            def _(g):
                s16 = ebuf[pl.ds(g * 16, 16)]
                d16 = ebuf[pl.ds(CHUNK + g * 16, 16)]
                a16 = plsc.bitcast(ebuf[pl.ds(2 * CHUNK + g * 16, 16)],
                                   jnp.float32)
                for d in range(DPT):
                    hv = plsc.load_gather(hcols, [s16 + d * NP])
                    msg = jnp.maximum(hv + (a16 * ewd[d] + ebd[d]), 0.0) + EPS_GEN
                    ex = jnp.exp(msg * tvec)
                    plsc.addupdate_scatter(numv, [d16 + d * NP], msg * ex)
                    plsc.addupdate_scatter(denv, [d16 + d * NP], ex)

        pltpu.async_copy(e_hbm.at[pl.ds(0, 3 * CHUNK)], eb0, sem0)
        pltpu.async_copy(e_hbm.at[pl.ds(3 * CHUNK, 3 * CHUNK)], eb1, sem1)

        def pair_body(p, carry):
            for b in range(2):
                c = 2 * p + b
                pltpu.make_async_copy(
                    e_hbm.at[pl.ds(0, 3 * CHUNK)], ebufs[b], sems[b]).wait()
                process(ebufs[b])

                @pl.when(c + 2 < NCHUNK)
                def _():
                    pltpu.async_copy(
                        e_hbm.at[pl.ds((c + 2) * 3 * CHUNK, 3 * CHUNK)],
                        ebufs[b], sems[b])
            return carry

        lax.fori_loop(0, NCHUNK // 2, pair_body, 0)

        pltpu.sync_copy(numv, num_hbm.at[pl.ds(base, DPT * NP)])
        pltpu.sync_copy(denv, den_hbm.at[pl.ds(base, DPT * NP)])

    return k(h_flat, epacked, pvec)


NT = 1024
GRID = NP // NT


def _ln_cols(y, g, b):
    mu = jnp.mean(y, axis=0, keepdims=True)
    var = jnp.mean((y - mu) ** 2, axis=0, keepdims=True)
    return (y - mu) / jnp.sqrt(var + LN_EPS) * g + b


def _encode_body(x_ref, nw_ref, nb_ref, o_ref):
    o_ref[...] = nw_ref[...] * x_ref[...] + nb_ref[...]


def _tc_encode(x2, nw, nb):
    return pl.pallas_call(
        _encode_body,
        grid=(GRID,),
        in_specs=[
            pl.BlockSpec((1, NT), lambda i: (0, i)),
            pl.BlockSpec((H, 1), lambda i: (0, 0)),
            pl.BlockSpec((H, 1), lambda i: (0, 0)),
        ],
        out_specs=pl.BlockSpec((H, NT), lambda i: (0, i)),
        out_shape=jax.ShapeDtypeStruct((H, NP), jnp.float32),
    )(x2, nw, nb)


def _layer_body(residual, num_ref, den_ref, hn_ref, hp_ref, w1_ref, b1_ref,
                g1_ref, be1_ref, w2_ref, b2_ref, gn_ref, bn_ref,
                ho_ref, hn2_ref):
    agg = num_ref[...] / (den_ref[...] + 1e-16)
    outt = agg + hn_ref[...]
    y1 = jnp.dot(w1_ref[...], outt, preferred_element_type=jnp.float32)
    y1 = y1 + b1_ref[...]
    y1 = jnp.maximum(_ln_cols(y1, g1_ref[...], be1_ref[...]), 0.0)
    y2 = jnp.dot(w2_ref[...], y1, preferred_element_type=jnp.float32)
    y2 = y2 + b2_ref[...]
    h_new = hp_ref[...] + y2 if residual else y2
    ho_ref[...] = h_new
    hn2_ref[...] = jnp.maximum(_ln_cols(h_new, gn_ref[...], bn_ref[...]), 0.0)


def _tc_layer(num2, den2, hn2, hp2, w1t, b1c, g1c, be1c, w2t, b2c, gnc, bnc,
              residual):
    big = pl.BlockSpec((H, NT), lambda i: (0, i))
    return pl.pallas_call(
        functools.partial(_layer_body, residual),
        grid=(GRID,),
        in_specs=[
            big, big, big, big,
            pl.BlockSpec((2 * H, H), lambda i: (0, 0)),
            pl.BlockSpec((2 * H, 1), lambda i: (0, 0)),
            pl.BlockSpec((2 * H, 1), lambda i: (0, 0)),
            pl.BlockSpec((2 * H, 1), lambda i: (0, 0)),
            pl.BlockSpec((H, 2 * H), lambda i: (0, 0)),
            pl.BlockSpec((H, 1), lambda i: (0, 0)),
            pl.BlockSpec((H, 1), lambda i: (0, 0)),
            pl.BlockSpec((H, 1), lambda i: (0, 0)),
        ],
        out_specs=[big, big],
        out_shape=[
            jax.ShapeDtypeStruct((H, NP), jnp.float32),
            jax.ShapeDtypeStruct((H, NP), jnp.float32),
        ],
    )(num2, den2, hn2, hp2, w1t, b1c, g1c, be1c, w2t, b2c, gnc, bnc)


def _last_body(num_ref, den_ref, hn_ref, hp_ref, w1_ref, b1_ref, g1_ref,
               be1_ref, w2_ref, b2_ref, gn_ref, bn_ref, lw_ref, sc_ref):
    agg = num_ref[...] / (den_ref[...] + 1e-16)
    outt = agg + hn_ref[...]
    y1 = jnp.dot(w1_ref[...], outt, preferred_element_type=jnp.float32)
    y1 = y1 + b1_ref[...]
    y1 = jnp.maximum(_ln_cols(y1, g1_ref[...], be1_ref[...]), 0.0)
    y2 = jnp.dot(w2_ref[...], y1, preferred_element_type=jnp.float32)
    y2 = y2 + b2_ref[...]
    h_new = hp_ref[...] + y2
    hfin = jnp.maximum(_ln_cols(h_new, gn_ref[...], bn_ref[...]), 0.0)
    part = jnp.sum(hfin * lw_ref[...]).reshape(1, 1)

    @pl.when(pl.program_id(0) == 0)
    def _():
        sc_ref[...] = jnp.zeros((1, 1), jnp.float32)

    sc_ref[...] += part


def _tc_last(num2, den2, hn2, hp2, w1t, b1c, g1c, be1c, w2t, b2c, gnc, bnc,
             lwt):
    big = pl.BlockSpec((H, NT), lambda i: (0, i))
    return pl.pallas_call(
        _last_body,
        grid=(GRID,),
        in_specs=[
            big, big, big, big,
            pl.BlockSpec((2 * H, H), lambda i: (0, 0)),
            pl.BlockSpec((2 * H, 1), lambda i: (0, 0)),
            pl.BlockSpec((2 * H, 1), lambda i: (0, 0)),
            pl.BlockSpec((2 * H, 1), lambda i: (0, 0)),
            pl.BlockSpec((H, 2 * H), lambda i: (0, 0)),
            pl.BlockSpec((H, 1), lambda i: (0, 0)),
            pl.BlockSpec((H, 1), lambda i: (0, 0)),
            pl.BlockSpec((H, 1), lambda i: (0, 0)),
            big,
        ],
        out_specs=pl.BlockSpec((1, 1), lambda i: (0, 0)),
        out_shape=jax.ShapeDtypeStruct((1, 1), jnp.float32),
    )(num2, den2, hn2, hp2, w1t, b1c, g1c, be1c, w2t, b2c, gnc, bnc, lwt)


def kernel(x, edge_index, edge_attr, batch, params):
    src = edge_index[0].reshape(NCHUNK, 1, CHUNK)
    dst = edge_index[1].reshape(NCHUNK, 1, CHUNK)
    attr_i = lax.bitcast_convert_type(edge_attr, jnp.int32)
    attr_i = attr_i.reshape(NCHUNK, 1, CHUNK)
    epacked = jnp.concatenate([src, dst, attr_i], axis=1).reshape(3 * E)
    ew = params['edge_w'][0]
    eb = params['edge_b']
    nw = params['node_w'][0].reshape(H, 1)
    nb = params['node_b'].reshape(H, 1)
    lwt = jnp.pad(params['lin_w'].reshape(N, H).T, ((0, 0), (0, NP - N)))

    pvecs = [
        jnp.concatenate([ew, eb, jnp.broadcast_to(params['t'][i], (16,)),
                         jnp.zeros((PVLEN - 2 * H - 16,), jnp.float32)])
        for i in range(NLAYERS)
    ]
    w1t = [params['w1'][i].T for i in range(NLAYERS)]
    w2t = [params['w2'][i].T for i in range(NLAYERS)]
    b1c = [params['b1'][i].reshape(2 * H, 1) for i in range(NLAYERS)]
    g1c = [params['g1'][i].reshape(2 * H, 1) for i in range(NLAYERS)]
    be1c = [params['be1'][i].reshape(2 * H, 1) for i in range(NLAYERS)]
    b2c = [params['b2'][i].reshape(H, 1) for i in range(NLAYERS)]
    bg = [params['blk_g'][i].reshape(H, 1) for i in range(NLAYERS)]
    bb = [params['blk_b'][i].reshape(H, 1) for i in range(NLAYERS)]

    hT = _tc_encode(jnp.pad(x.reshape(1, N), ((0, 0), (0, NP - N))), nw, nb)

    h = hT
    hn = hT
    for i in range(NLAYERS):
        num_f, den_f = _sc_edge_pass(hn.reshape(H * NP), epacked, pvecs[i])
        num2 = num_f.reshape(H, NP)
        den2 = den_f.reshape(H, NP)
        if i < NLAYERS - 1:
            h, hn = _tc_layer(num2, den2, hn, h, w1t[i], b1c[i], g1c[i],
                              be1c[i], w2t[i], b2c[i], bg[i + 1], bb[i + 1],
                              residual=(i > 0))
        else:
            scal = _tc_last(num2, den2, hn, h, w1t[i], b1c[i], g1c[i],
                            be1c[i], w2t[i], b2c[i], bg[0], bb[0], lwt)

    return scal.reshape(1) + params['lin_b']


# speedup vs baseline: 11.0799x; 1.3494x over previous
import functools

import jax
import jax.numpy as jnp
from jax import lax
from jax.experimental import pallas as pl
from jax.experimental.pallas import tpu as pltpu
from jax.experimental.pallas import tpu_sc as plsc

H = 128
N = 10000
NP = 10240
E = 320000
NLAYERS = 4
EPS_GEN = 1e-7
LN_EPS = 1e-5

NWORKERS = 32
DPT = H // NWORKERS
CHUNK = 800
NCHUNK = E // CHUNK
GROUPS = CHUNK // 16
PVLEN = 3 * H


def _sc_edge_pass(h_flat, epacked, pvec):
    mesh = plsc.VectorSubcoreMesh(core_axis_name="c", subcore_axis_name="s")

    @functools.partial(
        pl.kernel,
        mesh=mesh,
        compiler_params=pltpu.CompilerParams(needs_layout_passes=False),
        out_type=[
            jax.ShapeDtypeStruct((H * NP,), jnp.float32),
            jax.ShapeDtypeStruct((H * NP,), jnp.float32),
        ],
        scratch_types=[
            pltpu.VMEM((DPT * NP,), jnp.float32),
            pltpu.VMEM((DPT * NP,), jnp.float32),
            pltpu.VMEM((DPT * NP,), jnp.float32),
            pltpu.VMEM((3 * CHUNK,), jnp.int32),
            pltpu.VMEM((3 * CHUNK,), jnp.int32),
            pltpu.VMEM((PVLEN,), jnp.float32),
            pltpu.SemaphoreType.DMA,
            pltpu.SemaphoreType.DMA,
        ],
    )
    def k(h_hbm, e_hbm, pv_hbm, num_hbm, den_hbm,
          hcols, numv, denv, eb0, eb1, pv, sem0, sem1):
        wid = lax.axis_index("s") * 2 + lax.axis_index("c")
        base = wid * (DPT * NP)
        pltpu.sync_copy(h_hbm.at[pl.ds(base, DPT * NP)], hcols)
        pltpu.sync_copy(pv_hbm, pv)

        zz = jnp.zeros((16,), jnp.float32)

        @plsc.parallel_loop(0, DPT * NP // 16, unroll=8)
        def _(j):
            numv[pl.ds(j * 16, 16)] = zz
            denv[pl.ds(j * 16, 16)] = zz

        zi = jnp.zeros((16,), jnp.int32)
        tvec = pv[pl.ds(2 * H, 16)]
        ewd = [plsc.load_gather(pv, [zi + (wid * DPT + d)]) for d in range(DPT)]
        ebd = [plsc.load_gather(pv, [zi + (H + wid * DPT + d)]) for d in range(DPT)]

        ebufs = (eb0, eb1)
        sems = (sem0, sem1)

        def process(ebuf):
            @plsc.parallel_loop(0, GROUPS, unroll=5)
            def _(g):
                s16 = ebuf[pl.ds(g * 16, 16)]
                d16 = ebuf[pl.ds(CHUNK + g * 16, 16)]
                a16 = plsc.bitcast(ebuf[pl.ds(2 * CHUNK + g * 16, 16)],
                                   jnp.float32)
                for d in range(DPT):
                    hv = plsc.load_gather(hcols, [s16 + d * NP])
                    msg = jnp.maximum(hv + (a16 * ewd[d] + ebd[d]), 0.0) + EPS_GEN
                    ex = jnp.exp(msg * tvec)
                    plsc.addupdate_scatter(numv, [d16 + d * NP], msg * ex)
                    plsc.addupdate_scatter(denv, [d16 + d * NP], ex)

        pltpu.async_copy(e_hbm.at[pl.ds(0, 3 * CHUNK)], eb0, sem0)
        pltpu.async_copy(e_hbm.at[pl.ds(3 * CHUNK, 3 * CHUNK)], eb1, sem1)

        def pair_body(p, carry):
            for b in range(2):
                c = 2 * p + b
                pltpu.make_async_copy(
                    e_hbm.at[pl.ds(0, 3 * CHUNK)], ebufs[b], sems[b]).wait()
                process(ebufs[b])

                @pl.when(c + 2 < NCHUNK)
                def _():
                    pltpu.async_copy(
                        e_hbm.at[pl.ds((c + 2) * 3 * CHUNK, 3 * CHUNK)],
                        ebufs[b], sems[b])
            return carry

        lax.fori_loop(0, NCHUNK // 2, pair_body, 0)

        pltpu.sync_copy(numv, num_hbm.at[pl.ds(base, DPT * NP)])
        pltpu.sync_copy(denv, den_hbm.at[pl.ds(base, DPT * NP)])

    return k(h_flat, epacked, pvec)


NT = 1024
GRID = NP // NT


def _ln_cols(y, g, b):
    mu = jnp.mean(y, axis=0, keepdims=True)
    var = jnp.mean((y - mu) ** 2, axis=0, keepdims=True)
    return (y - mu) / jnp.sqrt(var + LN_EPS) * g + b


def _encode_body(x_ref, nw_ref, nb_ref, o_ref):
    o_ref[...] = nw_ref[...] * x_ref[...] + nb_ref[...]


def _tc_encode(x2, nw, nb):
    return pl.pallas_call(
        _encode_body,
        grid=(GRID,),
        in_specs=[
            pl.BlockSpec((1, NT), lambda i: (0, i)),
            pl.BlockSpec((H, 1), lambda i: (0, 0)),
            pl.BlockSpec((H, 1), lambda i: (0, 0)),
        ],
        out_specs=pl.BlockSpec((H, NT), lambda i: (0, i)),
        out_shape=jax.ShapeDtypeStruct((H, NP), jnp.float32),
    )(x2, nw, nb)


def _layer_body(residual, num_ref, den_ref, hn_ref, hp_ref, w1_ref, b1_ref,
                g1_ref, be1_ref, w2_ref, b2_ref, gn_ref, bn_ref,
                ho_ref, hn2_ref):
    agg = num_ref[...] / (den_ref[...] + 1e-16)
    outt = agg + hn_ref[...]
    y1 = jnp.dot(w1_ref[...], outt, preferred_element_type=jnp.float32)
    y1 = y1 + b1_ref[...]
    y1 = jnp.maximum(_ln_cols(y1, g1_ref[...], be1_ref[...]), 0.0)
    y2 = jnp.dot(w2_ref[...], y1, preferred_element_type=jnp.float32)
    y2 = y2 + b2_ref[...]
    h_new = hp_ref[...] + y2 if residual else y2
    ho_ref[...] = h_new
    hn2_ref[...] = jnp.maximum(_ln_cols(h_new, gn_ref[...], bn_ref[...]), 0.0)


def _tc_layer(num2, den2, hn2, hp2, w1t, b1c, g1c, be1c, w2t, b2c, gnc, bnc,
              residual):
    big = pl.BlockSpec((H, NT), lambda i: (0, i))
    return pl.pallas_call(
        functools.partial(_layer_body, residual),
        grid=(GRID,),
        in_specs=[
            big, big, big, big,
            pl.BlockSpec((2 * H, H), lambda i: (0, 0)),
            pl.BlockSpec((2 * H, 1), lambda i: (0, 0)),
            pl.BlockSpec((2 * H, 1), lambda i: (0, 0)),
            pl.BlockSpec((2 * H, 1), lambda i: (0, 0)),
            pl.BlockSpec((H, 2 * H), lambda i: (0, 0)),
            pl.BlockSpec((H, 1), lambda i: (0, 0)),
            pl.BlockSpec((H, 1), lambda i: (0, 0)),
            pl.BlockSpec((H, 1), lambda i: (0, 0)),
        ],
        out_specs=[big, big],
        out_shape=[
            jax.ShapeDtypeStruct((H, NP), jnp.float32),
            jax.ShapeDtypeStruct((H, NP), jnp.float32),
        ],
    )(num2, den2, hn2, hp2, w1t, b1c, g1c, be1c, w2t, b2c, gnc, bnc)


def _last_body(num_ref, den_ref, hn_ref, hp_ref, w1_ref, b1_ref, g1_ref,
               be1_ref, w2_ref, b2_ref, gn_ref, bn_ref, lw_ref, sc_ref):
    agg = num_ref[...] / (den_ref[...] + 1e-16)
    outt = agg + hn_ref[...]
    y1 = jnp.dot(w1_ref[...], outt, preferred_element_type=jnp.float32)
    y1 = y1 + b1_ref[...]
    y1 = jnp.maximum(_ln_cols(y1, g1_ref[...], be1_ref[...]), 0.0)
    y2 = jnp.dot(w2_ref[...], y1, preferred_element_type=jnp.float32)
    y2 = y2 + b2_ref[...]
    h_new = hp_ref[...] + y2
    hfin = jnp.maximum(_ln_cols(h_new, gn_ref[...], bn_ref[...]), 0.0)
    part = jnp.sum(hfin * lw_ref[...]).reshape(1, 1)

    @pl.when(pl.program_id(0) == 0)
    def _():
        sc_ref[...] = jnp.zeros((1, 1), jnp.float32)

    sc_ref[...] += part


def _tc_last(num2, den2, hn2, hp2, w1t, b1c, g1c, be1c, w2t, b2c, gnc, bnc,
             lwt):
    big = pl.BlockSpec((H, NT), lambda i: (0, i))
    return pl.pallas_call(
        _last_body,
        grid=(GRID,),
        in_specs=[
            big, big, big, big,
            pl.BlockSpec((2 * H, H), lambda i: (0, 0)),
            pl.BlockSpec((2 * H, 1), lambda i: (0, 0)),
            pl.BlockSpec((2 * H, 1), lambda i: (0, 0)),
            pl.BlockSpec((2 * H, 1), lambda i: (0, 0)),
            pl.BlockSpec((H, 2 * H), lambda i: (0, 0)),
            pl.BlockSpec((H, 1), lambda i: (0, 0)),
            pl.BlockSpec((H, 1), lambda i: (0, 0)),
            pl.BlockSpec((H, 1), lambda i: (0, 0)),
            big,
        ],
        out_specs=pl.BlockSpec((1, 1), lambda i: (0, 0)),
        out_shape=jax.ShapeDtypeStruct((1, 1), jnp.float32),
    )(num2, den2, hn2, hp2, w1t, b1c, g1c, be1c, w2t, b2c, gnc, bnc, lwt)


def kernel(x, edge_index, edge_attr, batch, params):
    src = edge_index[0].reshape(NCHUNK, 1, CHUNK)
    dst = edge_index[1].reshape(NCHUNK, 1, CHUNK)
    attr_i = lax.bitcast_convert_type(edge_attr, jnp.int32)
    attr_i = attr_i.reshape(NCHUNK, 1, CHUNK)
    epacked = jnp.concatenate([src, dst, attr_i], axis=1).reshape(3 * E)
    ew = params['edge_w'][0]
    eb = params['edge_b']
    nw = params['node_w'][0].reshape(H, 1)
    nb = params['node_b'].reshape(H, 1)
    lwt = jnp.pad(params['lin_w'].reshape(N, H).T, ((0, 0), (0, NP - N)))

    pvecs = [
        jnp.concatenate([ew, eb, jnp.broadcast_to(params['t'][i], (16,)),
                         jnp.zeros((PVLEN - 2 * H - 16,), jnp.float32)])
        for i in range(NLAYERS)
    ]
    w1t = [params['w1'][i].T for i in range(NLAYERS)]
    w2t = [params['w2'][i].T for i in range(NLAYERS)]
    b1c = [params['b1'][i].reshape(2 * H, 1) for i in range(NLAYERS)]
    g1c = [params['g1'][i].reshape(2 * H, 1) for i in range(NLAYERS)]
    be1c = [params['be1'][i].reshape(2 * H, 1) for i in range(NLAYERS)]
    b2c = [params['b2'][i].reshape(H, 1) for i in range(NLAYERS)]
    bg = [params['blk_g'][i].reshape(H, 1) for i in range(NLAYERS)]
    bb = [params['blk_b'][i].reshape(H, 1) for i in range(NLAYERS)]

    hT = _tc_encode(jnp.pad(x.reshape(1, N), ((0, 0), (0, NP - N))), nw, nb)

    h = hT
    hn = hT
    for i in range(NLAYERS):
        num_f, den_f = _sc_edge_pass(hn.reshape(H * NP), epacked, pvecs[i])
        num2 = num_f.reshape(H, NP)
        den2 = den_f.reshape(H, NP)
        if i < NLAYERS - 1:
            h, hn = _tc_layer(num2, den2, hn, h, w1t[i], b1c[i], g1c[i],
                              be1c[i], w2t[i], b2c[i], bg[i + 1], bb[i + 1],
                              residual=(i > 0))
        else:
            scal = _tc_last(num2, den2, hn, h, w1t[i], b1c[i], g1c[i],
                            be1c[i], w2t[i], b2c[i], bg[0], bb[0], lwt)

    return scal.reshape(1) + params['lin_b']


# speedup vs baseline: 11.3540x; 1.0247x over previous
import functools

import jax
import jax.numpy as jnp
from jax import lax
from jax.experimental import pallas as pl
from jax.experimental.pallas import tpu as pltpu
from jax.experimental.pallas import tpu_sc as plsc

H = 128
N = 10000
NP = 10240
E = 320000
NLAYERS = 4
EPS_GEN = 1e-7
LN_EPS = 1e-5

NWORKERS = 32
DPT = H // NWORKERS
CHUNK = 800
NCHUNK = E // CHUNK
GROUPS = CHUNK // 16
PVLEN = 3 * H


def _sc_edge_pass(h_flat, epacked, pvec):
    mesh = plsc.VectorSubcoreMesh(core_axis_name="c", subcore_axis_name="s")

    @functools.partial(
        pl.kernel,
        mesh=mesh,
        compiler_params=pltpu.CompilerParams(needs_layout_passes=False),
        out_type=[
            jax.ShapeDtypeStruct((H * NP,), jnp.float32),
            jax.ShapeDtypeStruct((H * NP,), jnp.float32),
        ],
        scratch_types=[
            pltpu.VMEM((DPT * NP,), jnp.float32),
            pltpu.VMEM((DPT * NP,), jnp.float32),
            pltpu.VMEM((DPT * NP,), jnp.float32),
            pltpu.VMEM((3 * CHUNK,), jnp.int32),
            pltpu.VMEM((3 * CHUNK,), jnp.int32),
            pltpu.VMEM((PVLEN,), jnp.float32),
            pltpu.SemaphoreType.DMA,
            pltpu.SemaphoreType.DMA,
        ],
    )
    def k(h_hbm, e_hbm, pv_hbm, num_hbm, den_hbm,
          hcols, numv, denv, eb0, eb1, pv, sem0, sem1):
        wid = lax.axis_index("s") * 2 + lax.axis_index("c")
        base = wid * (DPT * NP)
        pltpu.sync_copy(h_hbm.at[pl.ds(base, DPT * NP)], hcols)
        pltpu.sync_copy(pv_hbm, pv)

        zz = jnp.zeros((16,), jnp.float32)

        @plsc.parallel_loop(0, DPT * NP // 16, unroll=8)
        def _(j):
            numv[pl.ds(j * 16, 16)] = zz
            denv[pl.ds(j * 16, 16)] = zz

        zi = jnp.zeros((16,), jnp.int32)
        tvec = pv[pl.ds(2 * H, 16)]
        ewd = [plsc.load_gather(pv, [zi + (wid * DPT + d)]) for d in range(DPT)]
        ebd = [plsc.load_gather(pv, [zi + (H + wid * DPT + d)]) for d in range(DPT)]

        ebufs = (eb0, eb1)
        sems = (sem0, sem1)

        hc_d = [hcols.at[pl.ds(d * NP, NP)] for d in range(DPT)]
        num_d = [numv.at[pl.ds(d * NP, NP)] for d in range(DPT)]
        den_d = [denv.at[pl.ds(d * NP, NP)] for d in range(DPT)]

        def process(ebuf):
            @plsc.parallel_loop(0, GROUPS, unroll=5)
            def _(g):
                s16 = ebuf[pl.ds(g * 16, 16)]
                d16 = ebuf[pl.ds(CHUNK + g * 16, 16)]
                a16 = plsc.bitcast(ebuf[pl.ds(2 * CHUNK + g * 16, 16)],
                                   jnp.float32)
                for d in range(DPT):
                    hv = plsc.load_gather(hc_d[d], [s16])
                    msg = jnp.maximum(hv + (a16 * ewd[d] + ebd[d]),
                                      0.0) + EPS_GEN
                    ex = jnp.exp(msg * tvec)
                    plsc.addupdate_scatter(num_d[d], [d16], msg * ex)
                    plsc.addupdate_scatter(den_d[d], [d16], ex)

        pltpu.async_copy(e_hbm.at[pl.ds(0, 3 * CHUNK)], eb0, sem0)
        pltpu.async_copy(e_hbm.at[pl.ds(3 * CHUNK, 3 * CHUNK)], eb1, sem1)

        def pair_body(p, carry):
            for b in range(2):
                c = 2 * p + b
                pltpu.make_async_copy(
                    e_hbm.at[pl.ds(0, 3 * CHUNK)], ebufs[b], sems[b]).wait()
                process(ebufs[b])

                @pl.when(c + 2 < NCHUNK)
                def _():
                    pltpu.async_copy(
                        e_hbm.at[pl.ds((c + 2) * 3 * CHUNK, 3 * CHUNK)],
                        ebufs[b], sems[b])
            return carry

        lax.fori_loop(0, NCHUNK // 2, pair_body, 0)

        pltpu.sync_copy(numv, num_hbm.at[pl.ds(base, DPT * NP)])
        pltpu.sync_copy(denv, den_hbm.at[pl.ds(base, DPT * NP)])

    return k(h_flat, epacked, pvec)


NT = 1024
GRID = NP // NT


def _ln_cols(y, g, b):
    mu = jnp.mean(y, axis=0, keepdims=True)
    var = jnp.mean((y - mu) ** 2, axis=0, keepdims=True)
    return (y - mu) / jnp.sqrt(var + LN_EPS) * g + b


def _encode_body(x_ref, nw_ref, nb_ref, o_ref):
    o_ref[...] = nw_ref[...] * x_ref[...] + nb_ref[...]


def _tc_encode(x2, nw, nb):
    return pl.pallas_call(
        _encode_body,
        grid=(GRID,),
        in_specs=[
            pl.BlockSpec((1, NT), lambda i: (0, i)),
            pl.BlockSpec((H, 1), lambda i: (0, 0)),
            pl.BlockSpec((H, 1), lambda i: (0, 0)),
        ],
        out_specs=pl.BlockSpec((H, NT), lambda i: (0, i)),
        out_shape=jax.ShapeDtypeStruct((H, NP), jnp.float32),
    )(x2, nw, nb)


def _layer_body(residual, num_ref, den_ref, hn_ref, hp_ref, w1_ref, b1_ref,
                g1_ref, be1_ref, w2_ref, b2_ref, gn_ref, bn_ref,
                ho_ref, hn2_ref):
    agg = num_ref[...] / (den_ref[...] + 1e-16)
    outt = agg + hn_ref[...]
    y1 = jnp.dot(w1_ref[...], outt, preferred_element_type=jnp.float32)
    y1 = y1 + b1_ref[...]
    y1 = jnp.maximum(_ln_cols(y1, g1_ref[...], be1_ref[...]), 0.0)
    y2 = jnp.dot(w2_ref[...], y1, preferred_element_type=jnp.float32)
    y2 = y2 + b2_ref[...]
    h_new = hp_ref[...] + y2 if residual else y2
    ho_ref[...] = h_new
    hn2_ref[...] = jnp.maximum(_ln_cols(h_new, gn_ref[...], bn_ref[...]), 0.0)


def _tc_layer(num2, den2, hn2, hp2, w1t, b1c, g1c, be1c, w2t, b2c, gnc, bnc,
              residual):
    big = pl.BlockSpec((H, NT), lambda i: (0, i))
    return pl.pallas_call(
        functools.partial(_layer_body, residual),
        grid=(GRID,),
        in_specs=[
            big, big, big, big,
            pl.BlockSpec((2 * H, H), lambda i: (0, 0)),
            pl.BlockSpec((2 * H, 1), lambda i: (0, 0)),
            pl.BlockSpec((2 * H, 1), lambda i: (0, 0)),
            pl.BlockSpec((2 * H, 1), lambda i: (0, 0)),
            pl.BlockSpec((H, 2 * H), lambda i: (0, 0)),
            pl.BlockSpec((H, 1), lambda i: (0, 0)),
            pl.BlockSpec((H, 1), lambda i: (0, 0)),
            pl.BlockSpec((H, 1), lambda i: (0, 0)),
        ],
        out_specs=[big, big],
        out_shape=[
            jax.ShapeDtypeStruct((H, NP), jnp.float32),
            jax.ShapeDtypeStruct((H, NP), jnp.float32),
        ],
    )(num2, den2, hn2, hp2, w1t, b1c, g1c, be1c, w2t, b2c, gnc, bnc)


def _last_body(num_ref, den_ref, hn_ref, hp_ref, w1_ref, b1_ref, g1_ref,
               be1_ref, w2_ref, b2_ref, gn_ref, bn_ref, lw_ref, sc_ref):
    agg = num_ref[...] / (den_ref[...] + 1e-16)
    outt = agg + hn_ref[...]
    y1 = jnp.dot(w1_ref[...], outt, preferred_element_type=jnp.float32)
    y1 = y1 + b1_ref[...]
    y1 = jnp.maximum(_ln_cols(y1, g1_ref[...], be1_ref[...]), 0.0)
    y2 = jnp.dot(w2_ref[...], y1, preferred_element_type=jnp.float32)
    y2 = y2 + b2_ref[...]
    h_new = hp_ref[...] + y2
    hfin = jnp.maximum(_ln_cols(h_new, gn_ref[...], bn_ref[...]), 0.0)
    part = jnp.sum(hfin * lw_ref[...]).reshape(1, 1)

    @pl.when(pl.program_id(0) == 0)
    def _():
        sc_ref[...] = jnp.zeros((1, 1), jnp.float32)

    sc_ref[...] += part


def _tc_last(num2, den2, hn2, hp2, w1t, b1c, g1c, be1c, w2t, b2c, gnc, bnc,
             lwt):
    big = pl.BlockSpec((H, NT), lambda i: (0, i))
    return pl.pallas_call(
        _last_body,
        grid=(GRID,),
        in_specs=[
            big, big, big, big,
            pl.BlockSpec((2 * H, H), lambda i: (0, 0)),
            pl.BlockSpec((2 * H, 1), lambda i: (0, 0)),
            pl.BlockSpec((2 * H, 1), lambda i: (0, 0)),
            pl.BlockSpec((2 * H, 1), lambda i: (0, 0)),
            pl.BlockSpec((H, 2 * H), lambda i: (0, 0)),
            pl.BlockSpec((H, 1), lambda i: (0, 0)),
            pl.BlockSpec((H, 1), lambda i: (0, 0)),
            pl.BlockSpec((H, 1), lambda i: (0, 0)),
            big,
        ],
        out_specs=pl.BlockSpec((1, 1), lambda i: (0, 0)),
        out_shape=jax.ShapeDtypeStruct((1, 1), jnp.float32),
    )(num2, den2, hn2, hp2, w1t, b1c, g1c, be1c, w2t, b2c, gnc, bnc, lwt)


def kernel(x, edge_index, edge_attr, batch, params):
    src = edge_index[0].reshape(NCHUNK, 1, CHUNK)
    dst = edge_index[1].reshape(NCHUNK, 1, CHUNK)
    attr_i = lax.bitcast_convert_type(edge_attr, jnp.int32)
    attr_i = attr_i.reshape(NCHUNK, 1, CHUNK)
    epacked = jnp.concatenate([src, dst, attr_i], axis=1).reshape(3 * E)
    ew = params['edge_w'][0]
    eb = params['edge_b']
    nw = params['node_w'][0].reshape(H, 1)
    nb = params['node_b'].reshape(H, 1)
    lwt = jnp.pad(params['lin_w'].reshape(N, H).T, ((0, 0), (0, NP - N)))

    pvecs = [
        jnp.concatenate([ew, eb, jnp.broadcast_to(params['t'][i], (16,)),
                         jnp.zeros((PVLEN - 2 * H - 16,), jnp.float32)])
        for i in range(NLAYERS)
    ]
    w1t = [params['w1'][i].T for i in range(NLAYERS)]
    w2t = [params['w2'][i].T for i in range(NLAYERS)]
    b1c = [params['b1'][i].reshape(2 * H, 1) for i in range(NLAYERS)]
    g1c = [params['g1'][i].reshape(2 * H, 1) for i in range(NLAYERS)]
    be1c = [params['be1'][i].reshape(2 * H, 1) for i in range(NLAYERS)]
    b2c = [params['b2'][i].reshape(H, 1) for i in range(NLAYERS)]
    bg = [params['blk_g'][i].reshape(H, 1) for i in range(NLAYERS)]
    bb = [params['blk_b'][i].reshape(H, 1) for i in range(NLAYERS)]

    hT = _tc_encode(jnp.pad(x.reshape(1, N), ((0, 0), (0, NP - N))), nw, nb)

    h = hT
    hn = hT
    for i in range(NLAYERS):
        num_f, den_f = _sc_edge_pass(hn.reshape(H * NP), epacked, pvecs[i])
        num2 = num_f.reshape(H, NP)
        den2 = den_f.reshape(H, NP)
        if i < NLAYERS - 1:
            h, hn = _tc_layer(num2, den2, hn, h, w1t[i], b1c[i], g1c[i],
                              be1c[i], w2t[i], b2c[i], bg[i + 1], bb[i + 1],
                              residual=(i > 0))
        else:
            scal = _tc_last(num2, den2, hn, h, w1t[i], b1c[i], g1c[i],
                            be1c[i], w2t[i], b2c[i], bg[0], bb[0], lwt)

    return scal.reshape(1) + params['lin_b']
